# baseline (speedup 1.0000x reference)
import functools

import jax
import jax.numpy as jnp
from jax import lax
from jax.experimental import pallas as pl
from jax.experimental.pallas import tpu as pltpu
from jax.experimental.pallas import tpu_sc as plsc

_NC, _NS, _L = 2, 16, 16
_NW = _NC * _NS

_VOCAB = 1000000
_D = 64
_B = 16384
_BPW = _B // _NW
_CHUNK = 128
_NCHUNK = _BPW // _CHUNK
_SEG = _D // _L


def _rsqrt(x):
    i = lax.bitcast_convert_type(x, jnp.int32)
    i = jnp.int32(0x5F3759DF) - lax.shift_right_logical(i, 1)
    y = lax.bitcast_convert_type(i, jnp.float32)
    for _ in range(3):
        y = y * (1.5 - 0.5 * x * y * y)
    return y


_GATHER_DNUMS = lax.GatherDimensionNumbers(
    offset_dims=(), collapsed_slice_dims=(0,), start_index_map=(0,))


def _shuffle(v, idx):
    return lax.gather(v, idx[:, None], _GATHER_DNUMS, slice_sizes=(1,),
                      mode=lax.GatherScatterMode.PROMISE_IN_BOUNDS)


_mesh = plsc.VectorSubcoreMesh(core_axis_name="c", subcore_axis_name="s")


@functools.partial(
    pl.kernel,
    out_type=jax.ShapeDtypeStruct((_NW, _D), jnp.float32),
    mesh=_mesh,
    compiler_params=pltpu.CompilerParams(use_tc_tiling_on_sc=False),
    scratch_types=[
        pltpu.VMEM((_BPW,), jnp.int32),
        pltpu.VMEM((2, _CHUNK, _D), jnp.float32),
        pltpu.VMEM((_D,), jnp.float32),
        pltpu.SemaphoreType.DMA,
        pltpu.SemaphoreType.DMA,
    ],
)
def _holo_partials(idx_hbm, table_hbm, out_hbm, idx_v, rows_v, acc_v,
                   sem0, sem1):
    wid = lax.axis_index("s") * _NC + lax.axis_index("c")
    base = wid * _BPW
    pltpu.sync_copy(idx_hbm.at[pl.ds(base, _BPW)], idx_v)

    sems = (sem0, sem1)
    copies = [None, None]
    copies[0] = pltpu.async_copy(
        table_hbm.at[idx_v.at[pl.ds(0, _CHUNK)]], rows_v.at[0], sems[0])

    acc = (jnp.zeros((_L,), jnp.float32),) * _SEG
    for c in range(_NCHUNK):
        if c + 1 < _NCHUNK:
            nb = (c + 1) % 2
            copies[nb] = pltpu.async_copy(
                table_hbm.at[idx_v.at[pl.ds((c + 1) * _CHUNK, _CHUNK)]],
                rows_v.at[nb], sems[nb])
        copies[c % 2].wait()
        buf = rows_v.at[c % 2]

        def body(r, acc):
            segs = [buf[r, pl.ds(j * _L, _L)] for j in range(_SEG)]
            sq = segs[0] * segs[0]
            for j in range(1, _SEG):
                sq = sq + segs[j] * segs[j]
            lane = lax.iota(jnp.int32, _L)
            for s in (8, 4, 2, 1):
                sq = sq + _shuffle(sq, lane ^ s)
            rs = _rsqrt(sq)
            return tuple(a + s * rs for a, s in zip(acc, segs))

        acc = lax.fori_loop(0, _CHUNK, body, acc)

    for j in range(_SEG):
        acc_v[pl.ds(j * _L, _L)] = acc[j]
    pltpu.sync_copy(acc_v, out_hbm.at[wid])


def kernel(input_indices, weights):
    partials = _holo_partials(input_indices.astype(jnp.int32), weights)
    return jnp.sum(partials, axis=0, keepdims=True)


# speedup vs baseline: 1.0002x; 1.0002x over previous
import functools

import jax
import jax.numpy as jnp
from jax import lax
from jax.experimental import pallas as pl
from jax.experimental.pallas import tpu as pltpu
from jax.experimental.pallas import tpu_sc as plsc

_NC, _NS, _L = 2, 16, 16
_NW = _NC * _NS

_VOCAB = 1000000
_D = 64
_B = 16384
_BPW = _B // _NW
_CHUNK = 128
_NCHUNK = _BPW // _CHUNK
_SEG = _D // _L
_PD = 2 * _D


def _rsqrt(x):
    i = lax.bitcast_convert_type(x, jnp.int32)
    i = jnp.int32(0x5F3759DF) - lax.shift_right_logical(i, 1)
    y = lax.bitcast_convert_type(i, jnp.float32)
    for _ in range(3):
        y = y * (1.5 - 0.5 * x * y * y)
    return y


_GATHER_DNUMS = lax.GatherDimensionNumbers(
    offset_dims=(), collapsed_slice_dims=(0,), start_index_map=(0,))


def _shuffle(v, idx):
    return lax.gather(v, idx[:, None], _GATHER_DNUMS, slice_sizes=(1,),
                      mode=lax.GatherScatterMode.PROMISE_IN_BOUNDS)


_mesh = plsc.VectorSubcoreMesh(core_axis_name="c", subcore_axis_name="s")


@functools.partial(
    pl.kernel,
    out_type=jax.ShapeDtypeStruct((_NW, _D), jnp.float32),
    mesh=_mesh,
    scratch_types=[
        pltpu.VMEM((_BPW,), jnp.int32),
        pltpu.VMEM((_BPW,), jnp.int32),
        pltpu.VMEM((2, _CHUNK, _PD), jnp.float32),
        pltpu.VMEM((_D,), jnp.float32),
        pltpu.SemaphoreType.DMA,
        pltpu.SemaphoreType.DMA,
    ],
)
def _holo_partials(idx_hbm, wp_hbm, out_hbm, idx_v, idx2_v, rows_v,
                   acc_v, sem0, sem1):
    wid = lax.axis_index("s") * _NC + lax.axis_index("c")
    base = wid * _BPW
    pltpu.sync_copy(idx_hbm.at[pl.ds(base, _BPW)], idx_v)

    def half_idx(k, carry):
        sl = pl.ds(k * _L, _L)
        idx2_v[sl] = lax.shift_right_logical(idx_v[sl], 1)
        return carry

    lax.fori_loop(0, _BPW // _L, half_idx, 0)

    sems = (sem0, sem1)
    copies = [None, None]
    copies[0] = pltpu.async_copy(
        wp_hbm.at[idx2_v.at[pl.ds(0, _CHUNK)]], rows_v.at[0], sems[0])

    acc = (jnp.zeros((_L,), jnp.float32),) * _SEG
    lane = lax.iota(jnp.int32, _L)
    for c in range(_NCHUNK):
        if c + 1 < _NCHUNK:
            nb = (c + 1) % 2
            copies[nb] = pltpu.async_copy(
                wp_hbm.at[idx2_v.at[pl.ds((c + 1) * _CHUNK, _CHUNK)]],
                rows_v.at[nb], sems[nb])
        copies[c % 2].wait()
        buf = rows_v.at[c % 2]

        def body(r, acc):
            grp = lax.shift_right_logical(r, 4) * _L
            idxb = _shuffle(idx_v[pl.ds(c * _CHUNK + grp, _L)],
                            jnp.broadcast_to(r & (_L - 1), (_L,)))
            parf = (idxb & 1).astype(jnp.float32)
            segs = []
            for j in range(_SEG):
                lo = buf[r, pl.ds(j * _L, _L)]
                hi = buf[r, pl.ds(_D + j * _L, _L)]
                segs.append(lo + parf * (hi - lo))
            sq = segs[0] * segs[0]
            for j in range(1, _SEG):
                sq = sq + segs[j] * segs[j]
            for s in (8, 4, 2, 1):
                sq = sq + _shuffle(sq, lane ^ s)
            rs = _rsqrt(sq)
            return tuple(a + sg * rs for a, sg in zip(acc, segs))

        acc = lax.fori_loop(0, _CHUNK, body, acc)

    for j in range(_SEG):
        acc_v[pl.ds(j * _L, _L)] = acc[j]
    pltpu.sync_copy(acc_v, out_hbm.at[wid])


def kernel(input_indices, weights):
    wpairs = weights.reshape(_VOCAB // 2, _PD)
    partials = _holo_partials(input_indices.astype(jnp.int32), wpairs)
    return jnp.sum(partials, axis=0, keepdims=True)
